# Initial kernel scaffold; baseline (speedup 1.0000x reference)
import jax
import jax.numpy as jnp
from jax.experimental import pallas as pl


def kernel(inp, pos_s, pos_e, pe_s, pe_e, W1, b1, W2, b2, Wp, bp):
    raise NotImplementedError("write your pallas kernel here")


# speedup vs baseline: 1.6040x; 1.6040x over previous
import functools

import jax
import jax.numpy as jnp
from jax import lax
from jax.experimental import pallas as pl
from jax.experimental.pallas import tpu as pltpu
from jax.experimental.pallas import tpu_sc as plsc


_CHUNK = 64


def _sc_gather_pair(table_s, table_e, idx_s, idx_e):
    n = idx_s.shape[0]
    h = table_s.shape[1]
    info = plsc.get_sparse_core_info()
    nc, ns = info.num_cores, info.num_subcores
    nw = nc * ns
    per_w = n // nw
    nchunks = per_w // _CHUNK
    mesh = plsc.VectorSubcoreMesh(core_axis_name="core", subcore_axis_name="subcore")

    @functools.partial(
        pl.kernel,
        out_type=(
            jax.ShapeDtypeStruct((n, h), jnp.float32),
            jax.ShapeDtypeStruct((n, h), jnp.float32),
        ),
        mesh=mesh,
        scratch_types=[
            pltpu.VMEM((per_w,), jnp.int32),
            pltpu.VMEM((_CHUNK, h), jnp.float32),
            pltpu.SemaphoreType.DMA,
        ],
    )
    def gather_kernel(ts_hbm, te_hbm, is_hbm, ie_hbm, os_hbm, oe_hbm,
                      idx_v, rows_v, sem):
        wid = lax.axis_index("subcore") * nc + lax.axis_index("core")
        base = wid * per_w

        def one_table(t_hbm, i_hbm, o_hbm):
            pltpu.sync_copy(i_hbm.at[pl.ds(base, per_w)], idx_v)

            @pl.loop(0, nchunks)
            def _(c):
                off = c * _CHUNK
                pltpu.async_copy(
                    t_hbm.at[idx_v.at[pl.ds(off, _CHUNK)]], rows_v, sem
                ).wait()
                pltpu.sync_copy(rows_v, o_hbm.at[pl.ds(base + off, _CHUNK)])

        one_table(ts_hbm, is_hbm, os_hbm)
        one_table(te_hbm, ie_hbm, oe_hbm)

    return gather_kernel(table_s, table_e, idx_s, idx_e)


def _fold_body(w2_ref, wpb_ref, b2_ref, bp_ref, w2p_ref, bpr_ref):
    w2p_ref[...] = jnp.dot(
        w2_ref[...], wpb_ref[...], preferred_element_type=jnp.float32
    )
    bpr_ref[...] = (
        jnp.dot(b2_ref[...], wpb_ref[...], preferred_element_type=jnp.float32)
        + bp_ref[...]
    )


def _fold_weights(w2, wpb, b2, bp):
    h = w2.shape[0]
    return pl.pallas_call(
        _fold_body,
        out_shape=(
            jax.ShapeDtypeStruct((h, h), jnp.float32),
            jax.ShapeDtypeStruct((1, h), jnp.float32),
        ),
    )(w2, wpb, b2.reshape(1, h), bp.reshape(1, h))


_T = 256


def _fused_body(inp_ref, ps_ref, pe_ref, w1_ref, w2p_ref, wpa_ref, b1_ref,
                bpr_ref, out_ref):
    h = w2p_ref.shape[0]
    acc = jnp.dot(ps_ref[...], w1_ref[:h, :], preferred_element_type=jnp.float32)
    acc += jnp.dot(pe_ref[...], w1_ref[h:, :], preferred_element_type=jnp.float32)
    acc += b1_ref[...]
    acc = jnp.where(acc >= 0, acc, 0.01 * acc)
    out = jnp.dot(acc, w2p_ref[...], preferred_element_type=jnp.float32)
    out += jnp.dot(inp_ref[...], wpa_ref[...], preferred_element_type=jnp.float32)
    out_ref[...] = out + bpr_ref[...]


def _fused_chain(inp2, ps, pe, w1, w2p, wpa, b1, bpr):
    n, h = inp2.shape
    grid = (n // _T,)
    blk = lambda i: (i, 0)
    fixed = lambda i: (0, 0)
    return pl.pallas_call(
        _fused_body,
        grid=grid,
        in_specs=[
            pl.BlockSpec((_T, h), blk),
            pl.BlockSpec((_T, h), blk),
            pl.BlockSpec((_T, h), blk),
            pl.BlockSpec((2 * h, h), fixed),
            pl.BlockSpec((h, h), fixed),
            pl.BlockSpec((h, h), fixed),
            pl.BlockSpec((1, h), fixed),
            pl.BlockSpec((1, h), fixed),
        ],
        out_specs=pl.BlockSpec((_T, h), blk),
        out_shape=jax.ShapeDtypeStruct((n, h), jnp.float32),
        compiler_params=pltpu.CompilerParams(
            dimension_semantics=("arbitrary",),
        ),
    )(inp2, ps, pe, w1, w2p, wpa, b1, bpr)


def kernel(inp, pos_s, pos_e, pe_s, pe_e, W1, b1, W2, b2, Wp, bp):
    B, L, H = inp.shape
    n = B * L
    ps, pe_g = _sc_gather_pair(
        pe_s, pe_e, pos_s.reshape(n), pos_e.reshape(n)
    )
    w2p, bpr = _fold_weights(W2, Wp[H:], b2, bp)
    out = _fused_chain(
        inp.reshape(n, H), ps, pe_g, W1, w2p, Wp[:H], b1.reshape(1, H), bpr
    )
    return out.reshape(B, L, H)
